# Initial kernel scaffold; baseline (speedup 1.0000x reference)
import jax
import jax.numpy as jnp
from jax.experimental import pallas as pl


def kernel(arg0_1, arg1_1, arg2_1, arg3_1, arg4_1, arg5_1):
    raise NotImplementedError("write your pallas kernel here")


# speedup vs baseline: 2.4247x; 2.4247x over previous
import functools

import jax
import jax.numpy as jnp
from jax import lax
from jax.experimental import pallas as pl
from jax.experimental.pallas import tpu as pltpu
from jax.experimental.pallas import tpu_sc as plsc

_NUM_CORES = 2
_NUM_SUBCORES = 16
_NUM_WORKERS = _NUM_CORES * _NUM_SUBCORES
_LANES = 16

_B = 16384
_L = 50
_D = 64
_DV = _D // _LANES

_SAMPLES_PER_W = _B // _NUM_WORKERS
_CHUNK_SAMPLES = 16
_CHUNK_ROWS = _CHUNK_SAMPLES * _L
_GATHER_ROWS = 80
_GATHERS_PER_CHUNK = _CHUNK_ROWS // _GATHER_ROWS
_CHUNKS_PER_W = _SAMPLES_PER_W // _CHUNK_SAMPLES


def _sc_pool(table, idx_flat):
    mesh = plsc.VectorSubcoreMesh(
        core_axis_name="c", subcore_axis_name="s",
        num_cores=_NUM_CORES, num_subcores=_NUM_SUBCORES)

    @functools.partial(
        pl.kernel,
        out_type=jax.ShapeDtypeStruct((_B, _D), jnp.float32),
        mesh=mesh,
        scratch_types=[
            pltpu.VMEM((_CHUNK_ROWS,), jnp.int32),
            pltpu.VMEM((_CHUNK_ROWS, _D), jnp.float32),
            pltpu.VMEM((_CHUNK_SAMPLES, _D), jnp.float32),
            pltpu.SemaphoreType.DMA,
        ],
        compiler_params=pltpu.CompilerParams(use_tc_tiling_on_sc=False),
    )
    def sc_kernel(table_hbm, idx_hbm, out_hbm, idx_v, rows_v, pooled_v, sem):
        wid = lax.axis_index("s") * _NUM_CORES + lax.axis_index("c")
        w_row0 = wid * (_SAMPLES_PER_W * _L)
        w_samp0 = wid * _SAMPLES_PER_W

        def chunk_body(t, carry):
            row0 = w_row0 + t * _CHUNK_ROWS
            pltpu.sync_copy(idx_hbm.at[pl.ds(row0, _CHUNK_ROWS)], idx_v)
            copies = []
            for g in range(_GATHERS_PER_CHUNK):
                sl = pl.ds(g * _GATHER_ROWS, _GATHER_ROWS)
                copies.append(pltpu.async_copy(
                    table_hbm.at[idx_v.at[sl]], rows_v.at[sl], sem))
            for c in copies:
                c.wait()

            def sample_body(s, carry2):
                accs = [jnp.zeros((_LANES,), jnp.float32) for _ in range(_DV)]
                base = s * _L
                for j in range(_L):
                    r = base + j
                    for k in range(_DV):
                        accs[k] = accs[k] + rows_v[r, pl.ds(k * _LANES, _LANES)]
                for k in range(_DV):
                    pooled_v[s, pl.ds(k * _LANES, _LANES)] = accs[k]
                return carry2

            lax.fori_loop(0, _CHUNK_SAMPLES, sample_body, 0)
            pltpu.sync_copy(
                pooled_v,
                out_hbm.at[pl.ds(w_samp0 + t * _CHUNK_SAMPLES, _CHUNK_SAMPLES)])
            return carry

        lax.fori_loop(0, _CHUNKS_PER_W, chunk_body, 0)

    return sc_kernel(table, idx_flat)


def _mlp_body(x_ref, w1_ref, b1_ref, w2_ref, b2_ref, o_ref):
    x = x_ref[...] * (1.0 / _L)
    h = jnp.dot(x, w1_ref[...], preferred_element_type=jnp.float32)
    h = jnp.maximum(h + b1_ref[...], 0.0)
    o = jnp.dot(h, w2_ref[...], preferred_element_type=jnp.float32)
    o_ref[...] = o + b2_ref[...]


def _tc_mlp(pooled, w1t, b1, w2t, b2):
    bm = 2048
    h1 = w1t.shape[1]
    h2 = w2t.shape[1]
    return pl.pallas_call(
        _mlp_body,
        grid=(_B // bm,),
        in_specs=[
            pl.BlockSpec((bm, _D), lambda i: (i, 0)),
            pl.BlockSpec((_D, h1), lambda i: (0, 0)),
            pl.BlockSpec((1, h1), lambda i: (0, 0)),
            pl.BlockSpec((h1, h2), lambda i: (0, 0)),
            pl.BlockSpec((1, h2), lambda i: (0, 0)),
        ],
        out_specs=pl.BlockSpec((bm, h2), lambda i: (i, 0)),
        out_shape=jax.ShapeDtypeStruct((_B, h2), jnp.float32),
    )(pooled, w1t, b1, w2t, b2)


def kernel(arg0_1, arg1_1, arg2_1, arg3_1, arg4_1, arg5_1):
    idx_flat = arg1_1.reshape(-1)
    pooled = _sc_pool(arg0_1, idx_flat)
    w1t = arg2_1.T
    w2t = arg4_1.T
    b1 = arg3_1.reshape(1, -1)
    b2 = arg5_1.reshape(1, -1)
    out = _tc_mlp(pooled, w1t, b1, w2t, b2)
    return (out,)
